# baseline (speedup 1.0000x reference)
import jax
import jax.numpy as jnp
from jax import lax
from jax.experimental import pallas as pl
from jax.experimental.pallas import tpu as pltpu
from jax.experimental.pallas import tpu_sc as plsc

_NC = 2
_NS = 16
_NW = _NC * _NS

_CHUNK = 32
_NBUF = 4


def _make_gather(vocab, hidden, n_chunks, seq):
    mesh = plsc.VectorSubcoreMesh(core_axis_name="c", subcore_axis_name="s")
    b_per_w = n_chunks * _CHUNK
    w_per_row = seq // b_per_w

    @pl.kernel(
        out_type=jax.ShapeDtypeStruct((_NW * b_per_w, hidden), jnp.float32),
        mesh=mesh,
        scratch_types=[
            pltpu.VMEM((b_per_w,), jnp.int32),
            pltpu.VMEM((_NBUF, _CHUNK, hidden), jnp.float32),
        ] + [pltpu.SemaphoreType.DMA] * (2 * _NBUF),
    )
    def gather(idx_hbm, table_hbm, out_hbm, idx_v, rows_v, *sems):
        wid = lax.axis_index("s") * _NC + lax.axis_index("c")
        pltpu.sync_copy(
            idx_hbm.at[wid // w_per_row,
                       pl.ds((wid % w_per_row) * b_per_w, b_per_w)],
            idx_v)
        base = wid * b_per_w
        gsem = sems[:_NBUF]
        wsem = sems[_NBUF:]

        def issue_gather(c, b):
            pltpu.async_copy(
                table_hbm.at[idx_v.at[pl.ds(c * _CHUNK, _CHUNK)]],
                rows_v.at[b], gsem[b])

        def wait_gather(b):
            pltpu.make_async_copy(
                table_hbm.at[pl.ds(0, _CHUNK)], rows_v.at[b],
                gsem[b]).wait()

        def issue_write(c, b):
            pltpu.async_copy(
                rows_v.at[b], out_hbm.at[pl.ds(base + c * _CHUNK, _CHUNK)],
                wsem[b])

        def wait_write(b):
            pltpu.make_async_copy(
                rows_v.at[b], out_hbm.at[pl.ds(base, _CHUNK)],
                wsem[b]).wait()

        for b in range(_NBUF):
            issue_gather(b, b)

        @pl.loop(0, n_chunks - _NBUF, step=_NBUF)
        def _(g):
            for b in range(_NBUF):
                wait_gather(b)
                issue_write(g + b, b)
            for b in range(_NBUF):
                wait_write(b)
                issue_gather(g + _NBUF + b, b)

        for b in range(_NBUF):
            c = n_chunks - _NBUF + b
            wait_gather(b)
            issue_write(c, b)
        for b in range(_NBUF):
            wait_write(b)

    return gather


def kernel(input_ids, table):
    batch, seq = input_ids.shape
    vocab, hidden = table.shape
    total = batch * seq
    assert total % (_NW * _CHUNK) == 0
    n_chunks = total // (_NW * _CHUNK)
    assert n_chunks > _NBUF and (n_chunks - _NBUF) % _NBUF == 0
    out = _make_gather(vocab, hidden, n_chunks, seq)(
        input_ids.astype(jnp.int32), table)
    return out.reshape(batch, seq, hidden)


# speedup vs baseline: 1.0328x; 1.0328x over previous
import jax
import jax.numpy as jnp
from jax import lax
from jax.experimental import pallas as pl
from jax.experimental.pallas import tpu as pltpu
from jax.experimental.pallas import tpu_sc as plsc

_NC = 2
_NS = 16
_NW = _NC * _NS

_CHUNK = 16
_NBUF = 8


def _make_gather(vocab, hidden, n_chunks, seq):
    mesh = plsc.VectorSubcoreMesh(core_axis_name="c", subcore_axis_name="s")
    b_per_w = n_chunks * _CHUNK
    w_per_row = seq // b_per_w

    @pl.kernel(
        out_type=jax.ShapeDtypeStruct((_NW * b_per_w, hidden), jnp.float32),
        mesh=mesh,
        scratch_types=[
            pltpu.VMEM((b_per_w,), jnp.int32),
            pltpu.VMEM((_NBUF, _CHUNK, hidden), jnp.float32),
            pltpu.SemaphoreType.DMA((_NBUF,)),
            pltpu.SemaphoreType.DMA((_NBUF,)),
        ],
    )
    def gather(idx_hbm, table_hbm, out_hbm, idx_v, rows_v, gsems, wsems):
        wid = lax.axis_index("s") * _NC + lax.axis_index("c")
        pltpu.sync_copy(
            idx_hbm.at[wid // w_per_row,
                       pl.ds((wid % w_per_row) * b_per_w, b_per_w)],
            idx_v)
        base = wid * b_per_w
        gsem = [gsems.at[b] for b in range(_NBUF)]
        wsem = [wsems.at[b] for b in range(_NBUF)]

        def issue_gather(c, b):
            pltpu.async_copy(
                table_hbm.at[idx_v.at[pl.ds(c * _CHUNK, _CHUNK)]],
                rows_v.at[b], gsem[b])

        def wait_gather(b):
            pltpu.make_async_copy(
                table_hbm.at[pl.ds(0, _CHUNK)], rows_v.at[b],
                gsem[b]).wait()

        def issue_write(c, b):
            pltpu.async_copy(
                rows_v.at[b], out_hbm.at[pl.ds(base + c * _CHUNK, _CHUNK)],
                wsem[b])

        def wait_write(b):
            pltpu.make_async_copy(
                rows_v.at[b], out_hbm.at[pl.ds(base, _CHUNK)],
                wsem[b]).wait()

        for b in range(_NBUF):
            issue_gather(b, b)

        @pl.loop(0, n_chunks - _NBUF, step=_NBUF)
        def _(g):
            for b in range(_NBUF):
                wait_gather(b)
                issue_write(g + b, b)
            for b in range(_NBUF):
                wait_write(b)
                issue_gather(g + _NBUF + b, b)

        for b in range(_NBUF):
            c = n_chunks - _NBUF + b
            wait_gather(b)
            issue_write(c, b)
        for b in range(_NBUF):
            wait_write(b)

    return gather


def kernel(input_ids, table):
    batch, seq = input_ids.shape
    vocab, hidden = table.shape
    total = batch * seq
    assert total % (_NW * _CHUNK) == 0
    n_chunks = total // (_NW * _CHUNK)
    assert n_chunks > _NBUF and (n_chunks - _NBUF) % _NBUF == 0
    out = _make_gather(vocab, hidden, n_chunks, seq)(
        input_ids.astype(jnp.int32), table)
    return out.reshape(batch, seq, hidden)


# speedup vs baseline: 1.0360x; 1.0031x over previous
import jax
import jax.numpy as jnp
from jax import lax
from jax.experimental import pallas as pl
from jax.experimental.pallas import tpu as pltpu
from jax.experimental.pallas import tpu_sc as plsc

_NC = 2
_NS = 16
_NW = _NC * _NS

_CHUNK = 16
_NBUF = 8


def _make_gather(vocab, hidden, n_chunks, seq):
    mesh = plsc.VectorSubcoreMesh(core_axis_name="c", subcore_axis_name="s")
    b_per_w = n_chunks * _CHUNK
    w_per_row = seq // b_per_w

    @pl.kernel(
        out_type=jax.ShapeDtypeStruct((_NW * b_per_w, hidden), jnp.float32),
        mesh=mesh,
        scratch_types=[
            pltpu.VMEM((b_per_w,), jnp.int32),
            pltpu.VMEM((_NBUF * _CHUNK, hidden), jnp.float32),
            pltpu.SemaphoreType.DMA((_NBUF,)),
            pltpu.SemaphoreType.DMA((_NBUF // 2,)),
        ],
    )
    def gather(idx_hbm, table_hbm, out_hbm, idx_v, rows_v, gsems, wsems):
        wid = lax.axis_index("s") * _NC + lax.axis_index("c")
        pltpu.sync_copy(
            idx_hbm.at[wid // w_per_row,
                       pl.ds((wid % w_per_row) * b_per_w, b_per_w)],
            idx_v)
        base = wid * b_per_w
        gsem = [gsems.at[b] for b in range(_NBUF)]
        wsem = [wsems.at[p] for p in range(_NBUF // 2)]

        def issue_gather(c, b):
            pltpu.async_copy(
                table_hbm.at[idx_v.at[pl.ds(c * _CHUNK, _CHUNK)]],
                rows_v.at[pl.ds(b * _CHUNK, _CHUNK)], gsem[b])

        def wait_gather(b):
            pltpu.make_async_copy(
                table_hbm.at[pl.ds(0, _CHUNK)],
                rows_v.at[pl.ds(b * _CHUNK, _CHUNK)], gsem[b]).wait()

        def issue_write_pair(c, b):
            pltpu.async_copy(
                rows_v.at[pl.ds(b * _CHUNK, 2 * _CHUNK)],
                out_hbm.at[pl.ds(base + c * _CHUNK, 2 * _CHUNK)],
                wsem[b // 2])

        def wait_write_pair(b):
            pltpu.make_async_copy(
                rows_v.at[pl.ds(b * _CHUNK, 2 * _CHUNK)],
                out_hbm.at[pl.ds(base, 2 * _CHUNK)], wsem[b // 2]).wait()

        for b in range(_NBUF):
            issue_gather(b, b)

        @pl.loop(0, n_chunks - _NBUF, step=_NBUF)
        def _(g):
            for b in range(_NBUF):
                wait_gather(b)
                if b % 2 == 1:
                    issue_write_pair(g + b - 1, b - 1)
            for b in range(0, _NBUF, 2):
                wait_write_pair(b)
                issue_gather(g + _NBUF + b, b)
                issue_gather(g + _NBUF + b + 1, b + 1)

        for b in range(_NBUF):
            wait_gather(b)
            if b % 2 == 1:
                issue_write_pair(n_chunks - _NBUF + b - 1, b - 1)
        for b in range(0, _NBUF, 2):
            wait_write_pair(b)

    return gather


def kernel(input_ids, table):
    batch, seq = input_ids.shape
    vocab, hidden = table.shape
    total = batch * seq
    assert total % (_NW * _CHUNK) == 0
    n_chunks = total // (_NW * _CHUNK)
    assert n_chunks > _NBUF and (n_chunks - _NBUF) % _NBUF == 0
    out = _make_gather(vocab, hidden, n_chunks, seq)(
        input_ids.astype(jnp.int32), table)
    return out.reshape(batch, seq, hidden)
